# Initial kernel scaffold; baseline (speedup 1.0000x reference)
import jax
import jax.numpy as jnp
from jax.experimental import pallas as pl


def kernel(patch, patch_lib):
    raise NotImplementedError("write your pallas kernel here")


# speedup vs baseline: 1.2612x; 1.2612x over previous
import numpy as np
import jax
import jax.numpy as jnp
from jax.experimental import pallas as pl
from jax.experimental.pallas import tpu as pltpu

_F32 = jnp.float32
_BIG = np.int32(2**30)
_HI = jax.lax.Precision.HIGHEST


def _resize_mat(n_in: int, n_out: int) -> np.ndarray:
    scale = n_out / n_in
    sample = (np.arange(n_out, dtype=np.float64) + 0.5) / scale - 0.5
    x = np.abs(sample[:, None] - np.arange(n_in, dtype=np.float64)[None, :])
    w = np.maximum(0.0, 1.0 - x)
    tot = w.sum(axis=1, keepdims=True)
    w = np.where(np.abs(tot) > 1e-12, w / tot, 0.0)
    w *= ((sample >= -0.5) & (sample <= n_in - 0.5))[:, None]
    return w


def _blur_mat(n: int, sigma: float = 4.0) -> np.ndarray:
    radius = int(4 * sigma + 0.5)
    x = np.arange(-radius, radius + 1, dtype=np.float64)
    k = np.exp(-0.5 * (x / sigma) ** 2)
    k = k / k.sum()
    b = np.zeros((n, n), dtype=np.float64)
    for i in range(n):
        for t in range(-radius, radius + 1):
            j = i + t
            if j < 0:
                j = -j
            elif j >= n:
                j = 2 * n - 2 - j
            b[i, j] += k[t + radius]
    return b


_MMAT = np.asarray(_blur_mat(224) @ _resize_mat(28, 224), np.float32)
_MMAT_T = np.ascontiguousarray(_MMAT.T)


def _pick_block(m: int) -> int:
    for b in (2000, 1000, 200, 40, 8):
        if m % b == 0:
            return b
    return m


def _pass1_body(patch_t_ref, lib_ref, minval_ref, midx_ref, sidx_ref,
                sstar_ref, runmin_ref, runidx_ref):
    i = pl.program_id(0)
    nb = pl.num_programs(0)
    lib = lib_ref[...]
    pt = patch_t_ref[...]
    bsz = lib.shape[0]
    q = pt.shape[1]

    mm = jnp.dot(lib, pt, preferred_element_type=_F32, precision=_HI)
    b2 = jnp.sum(lib * lib, axis=1, keepdims=True)
    d2p = b2 - 2.0 * mm

    bmin = jnp.min(d2p, axis=0, keepdims=True)
    ii = jax.lax.broadcasted_iota(jnp.int32, d2p.shape, 0) + i * bsz
    bidx = jnp.min(jnp.where(d2p == bmin, ii, _BIG), axis=0, keepdims=True)

    @pl.when(i == 0)
    def _():
        runmin_ref[...] = jnp.full((1, q), jnp.inf, _F32)
        runidx_ref[...] = jnp.full((1, q), _BIG, jnp.int32)

    better = bmin < runmin_ref[...]
    runmin_new = jnp.where(better, bmin, runmin_ref[...])
    runidx_new = jnp.where(better, bidx, runidx_ref[...])
    runmin_ref[...] = runmin_new
    runidx_ref[...] = runidx_new

    @pl.when(i == nb - 1)
    def _():
        a2 = jnp.dot(jnp.ones((1, pt.shape[0]), _F32), pt * pt,
                     preferred_element_type=_F32, precision=_HI)
        mv = jnp.sqrt(jnp.maximum(runmin_new + a2, 1e-12))
        minval_ref[...] = mv
        sstar = jnp.max(mv, axis=1, keepdims=True)
        lane = jax.lax.broadcasted_iota(jnp.int32, (1, q), 1)
        sidx = jnp.min(jnp.where(mv == sstar, lane, _BIG), axis=1, keepdims=True)
        sidx_ref[...] = sidx
        sstar_ref[...] = sstar
        midx_ref[...] = jnp.min(jnp.where(lane == sidx, runidx_new, _BIG),
                                axis=1, keepdims=True)


def _insert3(state, bv, bc):
    v1, c1, v2, c2, v3, c3 = state
    lt1 = bv < v1
    lt2 = bv < v2
    lt3 = bv < v3
    nv1 = jnp.where(lt1, bv, v1)
    nc1 = jnp.where(lt1, bc, c1)
    nv2 = jnp.where(lt1, v1, jnp.where(lt2, bv, v2))
    nc2 = jnp.where(lt1, c1, jnp.where(lt2, bc, c2))
    nv3 = jnp.where(lt2, v2, jnp.where(lt3, bv, v3))
    nc3 = jnp.where(lt2, c2, jnp.where(lt3, bc, c3))
    return nv1, nc1, nv2, nc2, nv3, nc3


def _pass2_body(lib_ref, mq_t_ref, sstar_ref, s_ref, top_ref):
    i = pl.program_id(0)
    nb = pl.num_programs(0)
    lib = lib_ref[...]
    mq = mq_t_ref[...]
    bsz = lib.shape[0]

    mm = jnp.dot(lib, mq, preferred_element_type=_F32, precision=_HI)
    b2 = jnp.sum(lib * lib, axis=1, keepdims=True)
    q2 = jnp.sum(mq * mq, axis=0, keepdims=True)
    d2s = b2 - 2.0 * mm[:, 0:1] + q2[:, 0:1]
    d2t = b2 - 2.0 * mm[:, 1:2] + q2[:, 1:2]

    @pl.when(i == 0)
    def _():
        top_ref[...] = jnp.full((1, 8), jnp.inf, _F32)

    t = top_ref[...]
    state = (t[:, 0:1], t[:, 1:2], t[:, 2:3], t[:, 3:4], t[:, 4:5], t[:, 5:6])

    ii = jax.lax.broadcasted_iota(jnp.int32, d2s.shape, 0)
    work = d2s
    for _ in range(3):
        bv = jnp.min(work, axis=0, keepdims=True)
        bi = jnp.min(jnp.where(work == bv, ii, _BIG), axis=0, keepdims=True)
        hit = ii == bi
        bc = jnp.min(jnp.where(hit, d2t, jnp.inf), axis=0, keepdims=True)
        state = _insert3(state, bv, bc)
        work = jnp.where(hit, jnp.inf, work)

    v1, c1, v2, c2, v3, c3 = state
    top_ref[...] = jnp.concatenate(
        [v1, c1, v2, c2, v3, c3, jnp.zeros((1, 2), _F32)], axis=1)

    @pl.when(i == nb - 1)
    def _():
        dd = jnp.sqrt(jnp.asarray(float(mq_t_ref.shape[0]), _F32))
        knn2 = jnp.sqrt(jnp.maximum(c2, 0.0))
        knn3 = jnp.sqrt(jnp.maximum(c3, 0.0))
        sstar = sstar_ref[...]
        w = 1.0 - jnp.exp(sstar / dd) / (jnp.exp(knn2 / dd) + jnp.exp(knn3 / dd))
        s_ref[...] = w * sstar


def _smap_body(sq_ref, m_ref, mt_ref, out_ref):
    tmp = jnp.dot(m_ref[...], sq_ref[...], preferred_element_type=_F32,
                  precision=_HI)
    out_ref[...] = jnp.dot(tmp, mt_ref[...], preferred_element_type=_F32,
                           precision=_HI)


def kernel(patch, patch_lib):
    q, d = patch.shape
    m = patch_lib.shape[0]
    bsz = _pick_block(m)
    nb = m // bsz

    patch_t = patch.T

    minval, midx, sidx, sstar = pl.pallas_call(
        _pass1_body,
        grid=(nb,),
        in_specs=[
            pl.BlockSpec((d, q), lambda i: (0, 0)),
            pl.BlockSpec((bsz, d), lambda i: (i, 0)),
        ],
        out_specs=[
            pl.BlockSpec((1, q), lambda i: (0, 0)),
            pl.BlockSpec((1, 1), lambda i: (0, 0)),
            pl.BlockSpec((1, 1), lambda i: (0, 0)),
            pl.BlockSpec((1, 1), lambda i: (0, 0)),
        ],
        out_shape=[
            jax.ShapeDtypeStruct((1, q), _F32),
            jax.ShapeDtypeStruct((1, 1), jnp.int32),
            jax.ShapeDtypeStruct((1, 1), jnp.int32),
            jax.ShapeDtypeStruct((1, 1), _F32),
        ],
        scratch_shapes=[
            pltpu.VMEM((1, q), _F32),
            pltpu.VMEM((1, q), jnp.int32),
        ],
    )(patch_t, patch_lib)

    mstar = jax.lax.dynamic_slice(patch_lib, (midx[0, 0], 0), (1, d))
    mtest = jax.lax.dynamic_slice(patch, (sidx[0, 0], 0), (1, d))
    mq_t = jnp.concatenate([mstar, mtest], axis=0).T

    s, _top = pl.pallas_call(
        _pass2_body,
        grid=(nb,),
        in_specs=[
            pl.BlockSpec((bsz, d), lambda i: (i, 0)),
            pl.BlockSpec((d, 2), lambda i: (0, 0)),
            pl.BlockSpec((1, 1), lambda i: (0, 0)),
        ],
        out_specs=[
            pl.BlockSpec((1, 1), lambda i: (0, 0)),
            pl.BlockSpec((1, 8), lambda i: (0, 0)),
        ],
        out_shape=[
            jax.ShapeDtypeStruct((1, 1), _F32),
            jax.ShapeDtypeStruct((1, 8), _F32),
        ],
    )(patch_lib, mq_t, sstar)

    fh = int(round(float(np.sqrt(q))))
    mmat = jnp.asarray(_MMAT)
    mmat_t = jnp.asarray(_MMAT_T)
    smap = pl.pallas_call(
        _smap_body,
        out_shape=jax.ShapeDtypeStruct((mmat.shape[0], mmat.shape[0]), _F32),
    )(minval.reshape(fh, fh), mmat, mmat_t)

    return s[0, 0], smap[None, None]


# speedup vs baseline: 2.0722x; 1.6430x over previous
import numpy as np
import jax
import jax.numpy as jnp
from jax.experimental import pallas as pl
from jax.experimental.pallas import tpu as pltpu

_F32 = jnp.float32
_BIG = np.int32(2**30)
_HI = jax.lax.Precision.HIGHEST
_MED = jax.lax.Precision.HIGH


def _resize_mat(n_in: int, n_out: int) -> np.ndarray:
    scale = n_out / n_in
    sample = (np.arange(n_out, dtype=np.float64) + 0.5) / scale - 0.5
    x = np.abs(sample[:, None] - np.arange(n_in, dtype=np.float64)[None, :])
    w = np.maximum(0.0, 1.0 - x)
    tot = w.sum(axis=1, keepdims=True)
    w = np.where(np.abs(tot) > 1e-12, w / tot, 0.0)
    w *= ((sample >= -0.5) & (sample <= n_in - 0.5))[:, None]
    return w


def _blur_mat(n: int, sigma: float = 4.0) -> np.ndarray:
    radius = int(4 * sigma + 0.5)
    x = np.arange(-radius, radius + 1, dtype=np.float64)
    k = np.exp(-0.5 * (x / sigma) ** 2)
    k = k / k.sum()
    b = np.zeros((n, n), dtype=np.float64)
    for i in range(n):
        for t in range(-radius, radius + 1):
            j = i + t
            if j < 0:
                j = -j
            elif j >= n:
                j = 2 * n - 2 - j
            b[i, j] += k[t + radius]
    return b


_MMAT = np.asarray(_blur_mat(224) @ _resize_mat(28, 224), np.float32)
_MMAT_T = np.ascontiguousarray(_MMAT.T)


def _pick_block(m: int) -> int:
    for b in (2000, 1000, 200, 40, 8):
        if m % b == 0:
            return b
    return m


def _pass1_body(patch_t_ref, pt_hi_ref, pt_lo_ref, lib_ref, minval_ref,
                midx_ref, sidx_ref, sstar_ref, runmin_ref, runidx_ref):
    i = pl.program_id(0)
    nb = pl.num_programs(0)
    lib = lib_ref[...]
    pt = patch_t_ref[...]
    bsz = lib.shape[0]
    q = pt.shape[1]

    lib_hi = lib.astype(jnp.bfloat16)
    lib_lo = (lib - lib_hi.astype(_F32)).astype(jnp.bfloat16)
    pt_hi = pt_hi_ref[...]
    pt_lo = pt_lo_ref[...]
    mm = (jnp.dot(lib_hi, pt_hi, preferred_element_type=_F32)
          + jnp.dot(lib_hi, pt_lo, preferred_element_type=_F32)
          + jnp.dot(lib_lo, pt_hi, preferred_element_type=_F32))
    b2 = jnp.sum(lib * lib, axis=1, keepdims=True)
    d2p = b2 - 2.0 * mm

    bmin = jnp.min(d2p, axis=0, keepdims=True)
    ii = jax.lax.broadcasted_iota(jnp.int32, d2p.shape, 0) + i * bsz
    bidx = jnp.min(jnp.where(d2p == bmin, ii, _BIG), axis=0, keepdims=True)

    @pl.when(i == 0)
    def _():
        runmin_ref[...] = jnp.full((1, q), jnp.inf, _F32)
        runidx_ref[...] = jnp.full((1, q), _BIG, jnp.int32)

    better = bmin < runmin_ref[...]
    runmin_new = jnp.where(better, bmin, runmin_ref[...])
    runidx_new = jnp.where(better, bidx, runidx_ref[...])
    runmin_ref[...] = runmin_new
    runidx_ref[...] = runidx_new

    @pl.when(i == nb - 1)
    def _():
        a2 = jnp.dot(jnp.ones((1, pt.shape[0]), _F32), pt * pt,
                     preferred_element_type=_F32, precision=_HI)
        mv = jnp.sqrt(jnp.maximum(runmin_new + a2, 1e-12))
        minval_ref[...] = mv
        sstar = jnp.max(mv, axis=1, keepdims=True)
        lane = jax.lax.broadcasted_iota(jnp.int32, (1, q), 1)
        sidx = jnp.min(jnp.where(mv == sstar, lane, _BIG), axis=1, keepdims=True)
        sidx_ref[...] = sidx
        sstar_ref[...] = sstar
        midx_ref[...] = jnp.min(jnp.where(lane == sidx, runidx_new, _BIG),
                                axis=1, keepdims=True)


def _insert3(state, bv, bc):
    v1, c1, v2, c2, v3, c3 = state
    lt1 = bv < v1
    lt2 = bv < v2
    lt3 = bv < v3
    nv1 = jnp.where(lt1, bv, v1)
    nc1 = jnp.where(lt1, bc, c1)
    nv2 = jnp.where(lt1, v1, jnp.where(lt2, bv, v2))
    nc2 = jnp.where(lt1, c1, jnp.where(lt2, bc, c2))
    nv3 = jnp.where(lt2, v2, jnp.where(lt3, bv, v3))
    nc3 = jnp.where(lt2, c2, jnp.where(lt3, bc, c3))
    return nv1, nc1, nv2, nc2, nv3, nc3


def _pass2_body(lib_ref, mq_t_ref, sstar_ref, s_ref, top_ref):
    i = pl.program_id(0)
    nb = pl.num_programs(0)
    lib = lib_ref[...]
    mq = mq_t_ref[...]
    bsz = lib.shape[0]

    ds = lib - mq[0:1, :]
    dt = lib - mq[1:2, :]
    d2s = jnp.sum(ds * ds, axis=1, keepdims=True)
    d2t = jnp.sum(dt * dt, axis=1, keepdims=True)

    @pl.when(i == 0)
    def _():
        top_ref[...] = jnp.full((1, 8), jnp.inf, _F32)

    t = top_ref[...]
    state = (t[:, 0:1], t[:, 1:2], t[:, 2:3], t[:, 3:4], t[:, 4:5], t[:, 5:6])

    ii = jax.lax.broadcasted_iota(jnp.int32, d2s.shape, 0)
    work = d2s
    for _ in range(3):
        bv = jnp.min(work, axis=0, keepdims=True)
        bi = jnp.min(jnp.where(work == bv, ii, _BIG), axis=0, keepdims=True)
        hit = ii == bi
        bc = jnp.min(jnp.where(hit, d2t, jnp.inf), axis=0, keepdims=True)
        state = _insert3(state, bv, bc)
        work = jnp.where(hit, jnp.inf, work)

    v1, c1, v2, c2, v3, c3 = state
    top_ref[...] = jnp.concatenate(
        [v1, c1, v2, c2, v3, c3, jnp.zeros((1, 2), _F32)], axis=1)

    @pl.when(i == nb - 1)
    def _():
        dd = jnp.sqrt(jnp.asarray(float(mq_t_ref.shape[1]), _F32))
        knn2 = jnp.sqrt(jnp.maximum(c2, 0.0))
        knn3 = jnp.sqrt(jnp.maximum(c3, 0.0))
        sstar = sstar_ref[...]
        w = 1.0 - jnp.exp(sstar / dd) / (jnp.exp(knn2 / dd) + jnp.exp(knn3 / dd))
        s_ref[...] = w * sstar


def _smap_body(sq_ref, m_ref, mt_ref, out_ref):
    tmp = jnp.dot(m_ref[...], sq_ref[...], preferred_element_type=_F32,
                  precision=_HI)
    out_ref[...] = jnp.dot(tmp, mt_ref[...], preferred_element_type=_F32,
                           precision=_HI)


def kernel(patch, patch_lib):
    q, d = patch.shape
    m = patch_lib.shape[0]
    bsz = _pick_block(m)
    nb = m // bsz

    patch_t = patch.T
    pt_hi = patch_t.astype(jnp.bfloat16)
    pt_lo = (patch_t - pt_hi.astype(_F32)).astype(jnp.bfloat16)

    minval, midx, sidx, sstar = pl.pallas_call(
        _pass1_body,
        grid=(nb,),
        in_specs=[
            pl.BlockSpec((d, q), lambda i: (0, 0)),
            pl.BlockSpec((d, q), lambda i: (0, 0)),
            pl.BlockSpec((d, q), lambda i: (0, 0)),
            pl.BlockSpec((bsz, d), lambda i: (i, 0)),
        ],
        out_specs=[
            pl.BlockSpec((1, q), lambda i: (0, 0)),
            pl.BlockSpec((1, 1), lambda i: (0, 0)),
            pl.BlockSpec((1, 1), lambda i: (0, 0)),
            pl.BlockSpec((1, 1), lambda i: (0, 0)),
        ],
        out_shape=[
            jax.ShapeDtypeStruct((1, q), _F32),
            jax.ShapeDtypeStruct((1, 1), jnp.int32),
            jax.ShapeDtypeStruct((1, 1), jnp.int32),
            jax.ShapeDtypeStruct((1, 1), _F32),
        ],
        scratch_shapes=[
            pltpu.VMEM((1, q), _F32),
            pltpu.VMEM((1, q), jnp.int32),
        ],
    )(patch_t, pt_hi, pt_lo, patch_lib)

    mstar = jax.lax.dynamic_slice(patch_lib, (midx[0, 0], 0), (1, d))
    mtest = jax.lax.dynamic_slice(patch, (sidx[0, 0], 0), (1, d))
    mq = jnp.concatenate([mstar, mtest], axis=0)

    s, _top = pl.pallas_call(
        _pass2_body,
        grid=(nb,),
        in_specs=[
            pl.BlockSpec((bsz, d), lambda i: (i, 0)),
            pl.BlockSpec((2, d), lambda i: (0, 0)),
            pl.BlockSpec((1, 1), lambda i: (0, 0)),
        ],
        out_specs=[
            pl.BlockSpec((1, 1), lambda i: (0, 0)),
            pl.BlockSpec((1, 8), lambda i: (0, 0)),
        ],
        out_shape=[
            jax.ShapeDtypeStruct((1, 1), _F32),
            jax.ShapeDtypeStruct((1, 8), _F32),
        ],
    )(patch_lib, mq, sstar)

    fh = int(round(float(np.sqrt(q))))
    mmat = jnp.asarray(_MMAT)
    mmat_t = jnp.asarray(_MMAT_T)
    smap = pl.pallas_call(
        _smap_body,
        out_shape=jax.ShapeDtypeStruct((mmat.shape[0], mmat.shape[0]), _F32),
    )(minval.reshape(fh, fh), mmat, mmat_t)

    return s[0, 0], smap[None, None]
